# Initial kernel scaffold; baseline (speedup 1.0000x reference)
import jax
import jax.numpy as jnp
from jax.experimental import pallas as pl


def kernel(x_position, x_equity, edge_index_pe, edge_index_ep, edge_attr_pe, edge_attr_ep, W_msg_pe, b_msg_pe, W_msg_ep, b_msg_ep, W_root_pos, W_root_eq, W_clf_pos, W_clf_ast):
    raise NotImplementedError("write your pallas kernel here")


# speedup vs baseline: 2.1346x; 2.1346x over previous
import jax
import jax.numpy as jnp
from jax import lax
from jax.experimental import pallas as pl
from jax.experimental.pallas import tpu as pltpu
from jax.experimental.pallas import tpu_sc as plsc

N = 10000
E = 160000
D = 128
DE = 16
NSUB = 16
EPS = E // NSUB
CHUNK = 80
NCHUNK = EPS // CHUNK

_F32 = jnp.float32


def _prep_body(xp, xe, ea0, ea1, wx0, wx1, we0, we1, b0, b1,
               y0, y1, e0, e1):
    hi = jax.lax.Precision.HIGHEST
    y0[...] = jnp.dot(xp[...], wx0[...], precision=hi,
                      preferred_element_type=_F32)
    y1[...] = jnp.dot(xe[...], wx1[...], precision=hi,
                      preferred_element_type=_F32)
    e0[...] = jnp.dot(ea0[...], we0[...], precision=hi,
                      preferred_element_type=_F32) + b0[...]
    e1[...] = jnp.dot(ea1[...], we1[...], precision=hi,
                      preferred_element_type=_F32) + b1[...]


_PREP_GRID = 25
_NB = N // _PREP_GRID
_EB = E // _PREP_GRID

_prep = pl.pallas_call(
    _prep_body,
    grid=(_PREP_GRID,),
    in_specs=[
        pl.BlockSpec((_NB, D), lambda i: (i, 0)),
        pl.BlockSpec((_NB, D), lambda i: (i, 0)),
        pl.BlockSpec((_EB, DE), lambda i: (i, 0)),
        pl.BlockSpec((_EB, DE), lambda i: (i, 0)),
        pl.BlockSpec((D, D), lambda i: (0, 0)),
        pl.BlockSpec((D, D), lambda i: (0, 0)),
        pl.BlockSpec((DE, D), lambda i: (0, 0)),
        pl.BlockSpec((DE, D), lambda i: (0, 0)),
        pl.BlockSpec((1, D), lambda i: (0, 0)),
        pl.BlockSpec((1, D), lambda i: (0, 0)),
    ],
    out_specs=[
        pl.BlockSpec((_NB, D), lambda i: (i, 0)),
        pl.BlockSpec((_NB, D), lambda i: (i, 0)),
        pl.BlockSpec((_EB, D), lambda i: (i, 0)),
        pl.BlockSpec((_EB, D), lambda i: (i, 0)),
    ],
    out_shape=[
        jax.ShapeDtypeStruct((N, D), _F32),
        jax.ShapeDtypeStruct((N, D), _F32),
        jax.ShapeDtypeStruct((E, D), _F32),
        jax.ShapeDtypeStruct((E, D), _F32),
    ],
)


def _sc_body(y0, e0, s0, d0, y1, e1, s1, d1, iota, ones,
             sum0, cnt0, sum1, cnt1,
             acc, iv_s, iv_d, rows_v, emb_v, sem):
    cid = lax.axis_index("core")
    sid = lax.axis_index("subcore")
    base = pl.multiple_of(sid * 624, 8)
    zero16 = jnp.zeros((16,), _F32)

    def zero_acc():
        @pl.loop(0, CHUNK)
        def _(r):
            for c in range(8):
                rows_v[r, pl.ds(c * 16, 16)] = zero16

        for off in range(0, 640, CHUNK):
            pltpu.sync_copy(iota.at[pl.ds(base + off, CHUNK)], iv_s)
            pltpu.sync_copy(rows_v, acc.at[iv_s])

    def flush(out):
        for off in range(0, 640, CHUNK):
            pltpu.sync_copy(iota.at[pl.ds(base + off, CHUNK)], iv_s)
            pltpu.sync_copy(acc.at[iv_s], rows_v)
            pltpu.sync_copy(rows_v, out.at[pl.ds(base + off, CHUNK)])

    def msg_pass(tbl, emb, srcf, dstf):
        ebase = sid * EPS

        @pl.loop(0, NCHUNK)
        def _(j):
            eoff = pl.multiple_of(ebase + j * CHUNK, 8)
            pltpu.sync_copy(srcf.at[pl.ds(eoff, CHUNK)], iv_s)
            pltpu.sync_copy(dstf.at[pl.ds(eoff, CHUNK)], iv_d)
            pltpu.async_copy(tbl.at[iv_s], rows_v, sem).wait()
            pltpu.sync_copy(emb.at[pl.ds(eoff, CHUNK)], emb_v)

            @pl.loop(0, CHUNK)
            def _(r):
                for c in range(8):
                    sl = pl.ds(c * 16, 16)
                    rows_v[r, sl] = jnp.maximum(
                        rows_v[r, sl] + emb_v[r, sl], 0.0)

            pltpu.sync_copy(rows_v, acc.at[iv_d], add=True)

    def cnt_pass(dstf):
        ebase = sid * EPS
        pltpu.sync_copy(ones, emb_v)

        @pl.loop(0, NCHUNK)
        def _(j):
            eoff = pl.multiple_of(ebase + j * CHUNK, 8)
            pltpu.sync_copy(dstf.at[pl.ds(eoff, CHUNK)], iv_d)
            pltpu.sync_copy(emb_v, acc.at[iv_d], add=True)

    zero_acc()
    plsc.subcore_barrier()

    @pl.when(cid == 0)
    def _():
        msg_pass(y0, e0, s0, d0)

    @pl.when(cid == 1)
    def _():
        msg_pass(y1, e1, s1, d1)

    plsc.subcore_barrier()

    @pl.when(cid == 0)
    def _():
        flush(sum0)

    @pl.when(cid == 1)
    def _():
        flush(sum1)

    plsc.subcore_barrier()
    zero_acc()
    plsc.subcore_barrier()

    @pl.when(cid == 0)
    def _():
        cnt_pass(d0)

    @pl.when(cid == 1)
    def _():
        cnt_pass(d1)

    plsc.subcore_barrier()

    @pl.when(cid == 0)
    def _():
        flush(cnt0)

    @pl.when(cid == 1)
    def _():
        flush(cnt1)


_sc_agg = pl.kernel(
    _sc_body,
    out_type=[
        jax.ShapeDtypeStruct((N, D), _F32),
        jax.ShapeDtypeStruct((N, D), _F32),
        jax.ShapeDtypeStruct((N, D), _F32),
        jax.ShapeDtypeStruct((N, D), _F32),
    ],
    mesh=plsc.VectorSubcoreMesh(core_axis_name="core",
                                subcore_axis_name="subcore"),
    scratch_types=[
        pltpu.VMEM_SHARED((N, D), _F32),
        pltpu.VMEM((CHUNK,), jnp.int32),
        pltpu.VMEM((CHUNK,), jnp.int32),
        pltpu.VMEM((CHUNK, D), _F32),
        pltpu.VMEM((CHUNK, D), _F32),
        pltpu.SemaphoreType.DMA,
    ],
)


def _final_body(xp, sp, cp, wrp, wcp, xe, se, ce, wre, wce, op, oe):
    hi = jax.lax.Precision.HIGHEST

    def head(x, s, c, wr, wc, o):
        agg = s[...] / jnp.maximum(c[...], 1.0)
        out = jnp.dot(x[...], wr[...], precision=hi,
                      preferred_element_type=_F32) + agg
        o[...] = jnp.dot(jnp.maximum(out, 0.0), wc[...], precision=hi,
                         preferred_element_type=_F32)

    head(xp, sp, cp, wrp, wcp, op)
    head(xe, se, ce, wre, wce, oe)


_FIN_GRID = 10
_FB = N // _FIN_GRID

_final = pl.pallas_call(
    _final_body,
    grid=(_FIN_GRID,),
    in_specs=[
        pl.BlockSpec((_FB, D), lambda i: (i, 0)),
        pl.BlockSpec((_FB, D), lambda i: (i, 0)),
        pl.BlockSpec((_FB, D), lambda i: (i, 0)),
        pl.BlockSpec((D, D), lambda i: (0, 0)),
        pl.BlockSpec((D, D), lambda i: (0, 0)),
        pl.BlockSpec((_FB, D), lambda i: (i, 0)),
        pl.BlockSpec((_FB, D), lambda i: (i, 0)),
        pl.BlockSpec((_FB, D), lambda i: (i, 0)),
        pl.BlockSpec((D, D), lambda i: (0, 0)),
        pl.BlockSpec((D, D), lambda i: (0, 0)),
    ],
    out_specs=[
        pl.BlockSpec((_FB, D), lambda i: (i, 0)),
        pl.BlockSpec((_FB, D), lambda i: (i, 0)),
    ],
    out_shape=[
        jax.ShapeDtypeStruct((N, D), _F32),
        jax.ShapeDtypeStruct((N, D), _F32),
    ],
)


def kernel(x_position, x_equity, edge_index_pe, edge_index_ep,
           edge_attr_pe, edge_attr_ep, W_msg_pe, b_msg_pe, W_msg_ep,
           b_msg_ep, W_root_pos, W_root_eq, W_clf_pos, W_clf_ast):
    wx_pe, we_pe = W_msg_pe[:D], W_msg_pe[D:]
    wx_ep, we_ep = W_msg_ep[:D], W_msg_ep[D:]

    y_pe, y_ep, emb_pe, emb_ep = _prep(
        x_position, x_equity, edge_attr_pe, edge_attr_ep,
        wx_pe, wx_ep, we_pe, we_ep,
        b_msg_pe.reshape(1, D), b_msg_ep.reshape(1, D))

    iota = jnp.arange(N, dtype=jnp.int32)
    ones = jnp.ones((CHUNK, D), _F32)

    sum0, cnt0, sum1, cnt1 = _sc_agg(
        y_pe, emb_pe, edge_index_pe[0], edge_index_pe[1],
        y_ep, emb_ep, edge_index_ep[0], edge_index_ep[1],
        iota, ones)

    wc_pos = jnp.pad(W_clf_pos, ((0, 0), (0, D - W_clf_pos.shape[1])))
    wc_ast = jnp.pad(W_clf_ast, ((0, 0), (0, D - W_clf_ast.shape[1])))

    p_pos, p_eq = _final(x_position, sum1, cnt1, W_root_pos, wc_pos,
                         x_equity, sum0, cnt0, W_root_eq, wc_ast)

    nc = W_clf_pos.shape[1]
    return jnp.concatenate([p_pos[:, :nc], p_eq[:, :nc]], axis=0)


# speedup vs baseline: 2.3285x; 1.0909x over previous
import jax
import jax.numpy as jnp
from jax import lax
from jax.experimental import pallas as pl
from jax.experimental.pallas import tpu as pltpu
from jax.experimental.pallas import tpu_sc as plsc

N = 10000
E = 160000
D = 128
DE = 16
NSUB = 16
EPS = E // NSUB
CHUNK = 80
NCHUNK = EPS // CHUNK

_F32 = jnp.float32


def _prep_body(xp, xe, ea0, ea1, wx0, wx1, we0, we1, b0, b1,
               y0, y1, e0, e1):
    y0[...] = jnp.dot(xp[...], wx0[...], preferred_element_type=_F32)
    y1[...] = jnp.dot(xe[...], wx1[...], preferred_element_type=_F32)
    e0[...] = jnp.dot(ea0[...], we0[...], preferred_element_type=_F32) + b0[...]
    e1[...] = jnp.dot(ea1[...], we1[...], preferred_element_type=_F32) + b1[...]


_PREP_GRID = 25
_NB = N // _PREP_GRID
_EB = E // _PREP_GRID

_prep = pl.pallas_call(
    _prep_body,
    grid=(_PREP_GRID,),
    in_specs=[
        pl.BlockSpec((_NB, D), lambda i: (i, 0)),
        pl.BlockSpec((_NB, D), lambda i: (i, 0)),
        pl.BlockSpec((_EB, DE), lambda i: (i, 0)),
        pl.BlockSpec((_EB, DE), lambda i: (i, 0)),
        pl.BlockSpec((D, D), lambda i: (0, 0)),
        pl.BlockSpec((D, D), lambda i: (0, 0)),
        pl.BlockSpec((DE, D), lambda i: (0, 0)),
        pl.BlockSpec((DE, D), lambda i: (0, 0)),
        pl.BlockSpec((1, D), lambda i: (0, 0)),
        pl.BlockSpec((1, D), lambda i: (0, 0)),
    ],
    out_specs=[
        pl.BlockSpec((_NB, D), lambda i: (i, 0)),
        pl.BlockSpec((_NB, D), lambda i: (i, 0)),
        pl.BlockSpec((_EB, D), lambda i: (i, 0)),
        pl.BlockSpec((_EB, D), lambda i: (i, 0)),
    ],
    out_shape=[
        jax.ShapeDtypeStruct((N, D), _F32),
        jax.ShapeDtypeStruct((N, D), _F32),
        jax.ShapeDtypeStruct((E, D), _F32),
        jax.ShapeDtypeStruct((E, D), _F32),
    ],
)


def _sc_body(y0, e0, s0, d0, y1, e1, s1, d1, iota, ones,
             sum0, cnt0, sum1, cnt1,
             acc, iv_s, iv_d, rows_v, emb_v, sem):
    cid = lax.axis_index("core")
    sid = lax.axis_index("subcore")
    base = pl.multiple_of(sid * 624, 8)
    zero16 = jnp.zeros((16,), _F32)

    def zero_acc():
        @pl.loop(0, CHUNK)
        def _(r):
            for c in range(8):
                rows_v[r, pl.ds(c * 16, 16)] = zero16

        for off in range(0, 640, CHUNK):
            pltpu.sync_copy(iota.at[pl.ds(base + off, CHUNK)], iv_s)
            pltpu.sync_copy(rows_v, acc.at[iv_s])

    def flush(out):
        for off in range(0, 640, CHUNK):
            pltpu.sync_copy(iota.at[pl.ds(base + off, CHUNK)], iv_s)
            pltpu.sync_copy(acc.at[iv_s], rows_v)
            pltpu.sync_copy(rows_v, out.at[pl.ds(base + off, CHUNK)])

    def msg_pass(tbl, emb, srcf, dstf):
        ebase = sid * EPS

        @pl.loop(0, NCHUNK)
        def _(j):
            eoff = pl.multiple_of(ebase + j * CHUNK, 8)
            pltpu.sync_copy(srcf.at[pl.ds(eoff, CHUNK)], iv_s)
            pltpu.sync_copy(dstf.at[pl.ds(eoff, CHUNK)], iv_d)
            pltpu.async_copy(tbl.at[iv_s], rows_v, sem).wait()
            pltpu.sync_copy(emb.at[pl.ds(eoff, CHUNK)], emb_v)

            @pl.loop(0, CHUNK)
            def _(r):
                for c in range(8):
                    sl = pl.ds(c * 16, 16)
                    rows_v[r, sl] = jnp.maximum(
                        rows_v[r, sl] + emb_v[r, sl], 0.0)

            pltpu.sync_copy(rows_v, acc.at[iv_d], add=True)

    def cnt_pass(dstf):
        ebase = sid * EPS
        pltpu.sync_copy(ones, emb_v)

        @pl.loop(0, NCHUNK)
        def _(j):
            eoff = pl.multiple_of(ebase + j * CHUNK, 8)
            pltpu.sync_copy(dstf.at[pl.ds(eoff, CHUNK)], iv_d)
            pltpu.sync_copy(emb_v, acc.at[iv_d], add=True)

    zero_acc()
    plsc.subcore_barrier()

    @pl.when(cid == 0)
    def _():
        msg_pass(y0, e0, s0, d0)

    @pl.when(cid == 1)
    def _():
        msg_pass(y1, e1, s1, d1)

    plsc.subcore_barrier()

    @pl.when(cid == 0)
    def _():
        flush(sum0)

    @pl.when(cid == 1)
    def _():
        flush(sum1)

    plsc.subcore_barrier()
    zero_acc()
    plsc.subcore_barrier()

    @pl.when(cid == 0)
    def _():
        cnt_pass(d0)

    @pl.when(cid == 1)
    def _():
        cnt_pass(d1)

    plsc.subcore_barrier()

    @pl.when(cid == 0)
    def _():
        flush(cnt0)

    @pl.when(cid == 1)
    def _():
        flush(cnt1)


_sc_agg = pl.kernel(
    _sc_body,
    out_type=[
        jax.ShapeDtypeStruct((N, D), _F32),
        jax.ShapeDtypeStruct((N, D), _F32),
        jax.ShapeDtypeStruct((N, D), _F32),
        jax.ShapeDtypeStruct((N, D), _F32),
    ],
    mesh=plsc.VectorSubcoreMesh(core_axis_name="core",
                                subcore_axis_name="subcore"),
    scratch_types=[
        pltpu.VMEM_SHARED((N, D), _F32),
        pltpu.VMEM((CHUNK,), jnp.int32),
        pltpu.VMEM((CHUNK,), jnp.int32),
        pltpu.VMEM((CHUNK, D), _F32),
        pltpu.VMEM((CHUNK, D), _F32),
        pltpu.SemaphoreType.DMA,
    ],
)


def _final_body(xp, sp, cp, wrp, wcp, xe, se, ce, wre, wce, op, oe):
    def head(x, s, c, wr, wc, o):
        agg = s[...] / jnp.maximum(c[...], 1.0)
        out = jnp.dot(x[...], wr[...], preferred_element_type=_F32) + agg
        o[...] = jnp.dot(jnp.maximum(out, 0.0), wc[...],
                         preferred_element_type=_F32)

    head(xp, sp, cp, wrp, wcp, op)
    head(xe, se, ce, wre, wce, oe)


_FIN_GRID = 10
_FB = N // _FIN_GRID

_final = pl.pallas_call(
    _final_body,
    grid=(_FIN_GRID,),
    in_specs=[
        pl.BlockSpec((_FB, D), lambda i: (i, 0)),
        pl.BlockSpec((_FB, D), lambda i: (i, 0)),
        pl.BlockSpec((_FB, D), lambda i: (i, 0)),
        pl.BlockSpec((D, D), lambda i: (0, 0)),
        pl.BlockSpec((D, D), lambda i: (0, 0)),
        pl.BlockSpec((_FB, D), lambda i: (i, 0)),
        pl.BlockSpec((_FB, D), lambda i: (i, 0)),
        pl.BlockSpec((_FB, D), lambda i: (i, 0)),
        pl.BlockSpec((D, D), lambda i: (0, 0)),
        pl.BlockSpec((D, D), lambda i: (0, 0)),
    ],
    out_specs=[
        pl.BlockSpec((_FB, D), lambda i: (i, 0)),
        pl.BlockSpec((_FB, D), lambda i: (i, 0)),
    ],
    out_shape=[
        jax.ShapeDtypeStruct((N, D), _F32),
        jax.ShapeDtypeStruct((N, D), _F32),
    ],
)


def kernel(x_position, x_equity, edge_index_pe, edge_index_ep,
           edge_attr_pe, edge_attr_ep, W_msg_pe, b_msg_pe, W_msg_ep,
           b_msg_ep, W_root_pos, W_root_eq, W_clf_pos, W_clf_ast):
    wx_pe, we_pe = W_msg_pe[:D], W_msg_pe[D:]
    wx_ep, we_ep = W_msg_ep[:D], W_msg_ep[D:]

    y_pe, y_ep, emb_pe, emb_ep = _prep(
        x_position, x_equity, edge_attr_pe, edge_attr_ep,
        wx_pe, wx_ep, we_pe, we_ep,
        b_msg_pe.reshape(1, D), b_msg_ep.reshape(1, D))

    iota = jnp.arange(N, dtype=jnp.int32)
    ones = jnp.ones((CHUNK, D), _F32)

    sum0, cnt0, sum1, cnt1 = _sc_agg(
        y_pe, emb_pe, edge_index_pe[0], edge_index_pe[1],
        y_ep, emb_ep, edge_index_ep[0], edge_index_ep[1],
        iota, ones)

    wc_pos = jnp.pad(W_clf_pos, ((0, 0), (0, D - W_clf_pos.shape[1])))
    wc_ast = jnp.pad(W_clf_ast, ((0, 0), (0, D - W_clf_ast.shape[1])))

    p_pos, p_eq = _final(x_position, sum1, cnt1, W_root_pos, wc_pos,
                         x_equity, sum0, cnt0, W_root_eq, wc_ast)

    nc = W_clf_pos.shape[1]
    return jnp.concatenate([p_pos[:, :nc], p_eq[:, :nc]], axis=0)


# speedup vs baseline: 3.1987x; 1.3737x over previous
import jax
import jax.numpy as jnp
from jax import lax
from jax.experimental import pallas as pl
from jax.experimental.pallas import tpu as pltpu
from jax.experimental.pallas import tpu_sc as plsc

N = 10000
E = 160000
D = 128
DE = 16
NSUB = 16
EPS = E // NSUB
CHUNK = 80
NCHUNK = EPS // CHUNK

_F32 = jnp.float32


def _prep_body(xp, xe, ea0, ea1, wx0, wx1, we0, we1, b0, b1,
               y0, y1, e0, e1):
    y0[...] = jnp.dot(xp[...], wx0[...], preferred_element_type=_F32)
    y1[...] = jnp.dot(xe[...], wx1[...], preferred_element_type=_F32)
    e0[...] = jnp.dot(ea0[...], we0[...], preferred_element_type=_F32) + b0[...]
    e1[...] = jnp.dot(ea1[...], we1[...], preferred_element_type=_F32) + b1[...]


_PREP_GRID = 25
_NB = N // _PREP_GRID
_EB = E // _PREP_GRID

_prep = pl.pallas_call(
    _prep_body,
    grid=(_PREP_GRID,),
    in_specs=[
        pl.BlockSpec((_NB, D), lambda i: (i, 0)),
        pl.BlockSpec((_NB, D), lambda i: (i, 0)),
        pl.BlockSpec((_EB, DE), lambda i: (i, 0)),
        pl.BlockSpec((_EB, DE), lambda i: (i, 0)),
        pl.BlockSpec((D, D), lambda i: (0, 0)),
        pl.BlockSpec((D, D), lambda i: (0, 0)),
        pl.BlockSpec((DE, D), lambda i: (0, 0)),
        pl.BlockSpec((DE, D), lambda i: (0, 0)),
        pl.BlockSpec((1, D), lambda i: (0, 0)),
        pl.BlockSpec((1, D), lambda i: (0, 0)),
    ],
    out_specs=[
        pl.BlockSpec((_NB, D), lambda i: (i, 0)),
        pl.BlockSpec((_NB, D), lambda i: (i, 0)),
        pl.BlockSpec((_EB, D), lambda i: (i, 0)),
        pl.BlockSpec((_EB, D), lambda i: (i, 0)),
    ],
    out_shape=[
        jax.ShapeDtypeStruct((N, D), _F32),
        jax.ShapeDtypeStruct((N, D), _F32),
        jax.ShapeDtypeStruct((E, D), _F32),
        jax.ShapeDtypeStruct((E, D), _F32),
    ],
)


def _sc_body(y0, e0, s0, d0, y1, e1, s1, d1, iota, ones,
             sum0, cnt0, sum1, cnt1,
             acc, iv_s, iv_d, rows_v, emb_v,
             iv_s2, iv_d2, rows_v2, emb_v2, sem, sem2, sg, sg2):
    cid = lax.axis_index("core")
    sid = lax.axis_index("subcore")
    base = pl.multiple_of(sid * 624, 8)
    zero16 = jnp.zeros((16,), _F32)

    def zero_acc():
        @pl.loop(0, CHUNK)
        def _(r):
            for c in range(8):
                rows_v[r, pl.ds(c * 16, 16)] = zero16

        for off in range(0, 640, CHUNK):
            pltpu.sync_copy(iota.at[pl.ds(base + off, CHUNK)], iv_s)
            pltpu.sync_copy(rows_v, acc.at[iv_s])

    def flush(out):
        for off in range(0, 640, CHUNK):
            pltpu.sync_copy(iota.at[pl.ds(base + off, CHUNK)], iv_s)
            pltpu.sync_copy(acc.at[iv_s], rows_v)
            pltpu.sync_copy(rows_v, out.at[pl.ds(base + off, CHUNK)])

    def msg_pass(tbl, emb, srcf, dstf):
        ebase = sid * EPS
        sets = ((iv_s, iv_d, rows_v, emb_v, sem, sg),
                (iv_s2, iv_d2, rows_v2, emb_v2, sem2, sg2))

        def eoff_of(j):
            return pl.multiple_of(ebase + j * CHUNK, 8)

        def start_loads(j, s):
            ivs, ivd, rv, ev, sl, _ = s
            eoff = eoff_of(j)
            pltpu.async_copy(srcf.at[pl.ds(eoff, CHUNK)], ivs, sl)
            pltpu.async_copy(dstf.at[pl.ds(eoff, CHUNK)], ivd, sl)
            pltpu.async_copy(emb.at[pl.ds(eoff, CHUNK)], ev, sl)

        def process(j, s):
            ivs, ivd, rv, ev, sl, sgx = s
            eoff = eoff_of(j)
            pltpu.make_async_copy(srcf.at[pl.ds(eoff, CHUNK)], ivs, sl).wait()
            pltpu.make_async_copy(dstf.at[pl.ds(eoff, CHUNK)], ivd, sl).wait()
            pltpu.make_async_copy(emb.at[pl.ds(eoff, CHUNK)], ev, sl).wait()
            pltpu.async_copy(tbl.at[ivs], rv, sgx).wait()

            @pl.loop(0, CHUNK)
            def _(r):
                for c in range(8):
                    slc = pl.ds(c * 16, 16)
                    rv[r, slc] = jnp.maximum(rv[r, slc] + ev[r, slc], 0.0)

            pltpu.sync_copy(rv, acc.at[ivd], add=True)

        start_loads(0, sets[0])

        @pl.loop(0, NCHUNK // 2)
        def _(p):
            j0 = p * 2
            start_loads(j0 + 1, sets[1])
            process(j0, sets[0])
            start_loads(j0 + 2, sets[0])
            process(j0 + 1, sets[1])

        process(NCHUNK - 1, sets[0])

    def cnt_pass(dstf):
        ebase = sid * EPS
        pltpu.sync_copy(ones, emb_v)

        def eoff_of(j):
            return pl.multiple_of(ebase + j * CHUNK, 8)

        pltpu.async_copy(dstf.at[pl.ds(eoff_of(0), CHUNK)], iv_d, sem)

        @pl.loop(0, NCHUNK // 2)
        def _(p):
            j0 = p * 2
            pltpu.async_copy(dstf.at[pl.ds(eoff_of(j0 + 1), CHUNK)],
                             iv_d2, sem2)
            pltpu.make_async_copy(dstf.at[pl.ds(eoff_of(j0), CHUNK)],
                                  iv_d, sem).wait()
            pltpu.sync_copy(emb_v, acc.at[iv_d], add=True)
            pltpu.async_copy(dstf.at[pl.ds(eoff_of(j0 + 2), CHUNK)],
                             iv_d, sem)
            pltpu.make_async_copy(dstf.at[pl.ds(eoff_of(j0 + 1), CHUNK)],
                                  iv_d2, sem2).wait()
            pltpu.sync_copy(emb_v, acc.at[iv_d2], add=True)

        pltpu.make_async_copy(dstf.at[pl.ds(eoff_of(NCHUNK - 1), CHUNK)],
                              iv_d, sem).wait()
        pltpu.sync_copy(emb_v, acc.at[iv_d], add=True)

    zero_acc()
    plsc.subcore_barrier()

    @pl.when(cid == 0)
    def _():
        msg_pass(y0, e0, s0, d0)

    @pl.when(cid == 1)
    def _():
        msg_pass(y1, e1, s1, d1)

    plsc.subcore_barrier()

    @pl.when(cid == 0)
    def _():
        flush(sum0)

    @pl.when(cid == 1)
    def _():
        flush(sum1)

    plsc.subcore_barrier()
    zero_acc()
    plsc.subcore_barrier()

    @pl.when(cid == 0)
    def _():
        cnt_pass(d0)

    @pl.when(cid == 1)
    def _():
        cnt_pass(d1)

    plsc.subcore_barrier()

    @pl.when(cid == 0)
    def _():
        flush(cnt0)

    @pl.when(cid == 1)
    def _():
        flush(cnt1)


_sc_agg = pl.kernel(
    _sc_body,
    out_type=[
        jax.ShapeDtypeStruct((N, D), _F32),
        jax.ShapeDtypeStruct((N, D), _F32),
        jax.ShapeDtypeStruct((N, D), _F32),
        jax.ShapeDtypeStruct((N, D), _F32),
    ],
    mesh=plsc.VectorSubcoreMesh(core_axis_name="core",
                                subcore_axis_name="subcore"),
    scratch_types=[
        pltpu.VMEM_SHARED((N, D), _F32),
        pltpu.VMEM((CHUNK,), jnp.int32),
        pltpu.VMEM((CHUNK,), jnp.int32),
        pltpu.VMEM((CHUNK, D), _F32),
        pltpu.VMEM((CHUNK, D), _F32),
        pltpu.VMEM((CHUNK,), jnp.int32),
        pltpu.VMEM((CHUNK,), jnp.int32),
        pltpu.VMEM((CHUNK, D), _F32),
        pltpu.VMEM((CHUNK, D), _F32),
        pltpu.SemaphoreType.DMA,
        pltpu.SemaphoreType.DMA,
        pltpu.SemaphoreType.DMA,
        pltpu.SemaphoreType.DMA,
    ],
)


def _final_body(xp, sp, cp, wrp, wcp, xe, se, ce, wre, wce, op, oe):
    def head(x, s, c, wr, wc, o):
        agg = s[...] / jnp.maximum(c[...], 1.0)
        out = jnp.dot(x[...], wr[...], preferred_element_type=_F32) + agg
        o[...] = jnp.dot(jnp.maximum(out, 0.0), wc[...],
                         preferred_element_type=_F32)

    head(xp, sp, cp, wrp, wcp, op)
    head(xe, se, ce, wre, wce, oe)


_FIN_GRID = 10
_FB = N // _FIN_GRID

_final = pl.pallas_call(
    _final_body,
    grid=(_FIN_GRID,),
    in_specs=[
        pl.BlockSpec((_FB, D), lambda i: (i, 0)),
        pl.BlockSpec((_FB, D), lambda i: (i, 0)),
        pl.BlockSpec((_FB, D), lambda i: (i, 0)),
        pl.BlockSpec((D, D), lambda i: (0, 0)),
        pl.BlockSpec((D, D), lambda i: (0, 0)),
        pl.BlockSpec((_FB, D), lambda i: (i, 0)),
        pl.BlockSpec((_FB, D), lambda i: (i, 0)),
        pl.BlockSpec((_FB, D), lambda i: (i, 0)),
        pl.BlockSpec((D, D), lambda i: (0, 0)),
        pl.BlockSpec((D, D), lambda i: (0, 0)),
    ],
    out_specs=[
        pl.BlockSpec((_FB, D), lambda i: (i, 0)),
        pl.BlockSpec((_FB, D), lambda i: (i, 0)),
    ],
    out_shape=[
        jax.ShapeDtypeStruct((N, D), _F32),
        jax.ShapeDtypeStruct((N, D), _F32),
    ],
)


def kernel(x_position, x_equity, edge_index_pe, edge_index_ep,
           edge_attr_pe, edge_attr_ep, W_msg_pe, b_msg_pe, W_msg_ep,
           b_msg_ep, W_root_pos, W_root_eq, W_clf_pos, W_clf_ast):
    wx_pe, we_pe = W_msg_pe[:D], W_msg_pe[D:]
    wx_ep, we_ep = W_msg_ep[:D], W_msg_ep[D:]

    y_pe, y_ep, emb_pe, emb_ep = _prep(
        x_position, x_equity, edge_attr_pe, edge_attr_ep,
        wx_pe, wx_ep, we_pe, we_ep,
        b_msg_pe.reshape(1, D), b_msg_ep.reshape(1, D))

    iota = jnp.arange(N, dtype=jnp.int32)
    ones = jnp.ones((CHUNK, D), _F32)

    sum0, cnt0, sum1, cnt1 = _sc_agg(
        y_pe, emb_pe, edge_index_pe[0], edge_index_pe[1],
        y_ep, emb_ep, edge_index_ep[0], edge_index_ep[1],
        iota, ones)

    wc_pos = jnp.pad(W_clf_pos, ((0, 0), (0, D - W_clf_pos.shape[1])))
    wc_ast = jnp.pad(W_clf_ast, ((0, 0), (0, D - W_clf_ast.shape[1])))

    p_pos, p_eq = _final(x_position, sum1, cnt1, W_root_pos, wc_pos,
                         x_equity, sum0, cnt0, W_root_eq, wc_ast)

    nc = W_clf_pos.shape[1]
    return jnp.concatenate([p_pos[:, :nc], p_eq[:, :nc]], axis=0)


# speedup vs baseline: 3.3928x; 1.0607x over previous
import jax
import jax.numpy as jnp
from jax import lax
from jax.experimental import pallas as pl
from jax.experimental.pallas import tpu as pltpu
from jax.experimental.pallas import tpu_sc as plsc

N = 10000
E = 160000
D = 128
DE = 16
NSUB = 16
EPS = E // NSUB
CHUNK = 80
NCHUNK = EPS // CHUNK

_F32 = jnp.float32


def _prep_body(xp, xe, ea0, ea1, wx0, wx1, we0, we1, b0, b1,
               y0, y1, e0, e1):
    y0[...] = jnp.dot(xp[...], wx0[...], preferred_element_type=_F32)
    y1[...] = jnp.dot(xe[...], wx1[...], preferred_element_type=_F32)
    e0[...] = jnp.dot(ea0[...], we0[...], preferred_element_type=_F32) + b0[...]
    e1[...] = jnp.dot(ea1[...], we1[...], preferred_element_type=_F32) + b1[...]


_PREP_GRID = 25
_NB = N // _PREP_GRID
_EB = E // _PREP_GRID

_prep = pl.pallas_call(
    _prep_body,
    grid=(_PREP_GRID,),
    in_specs=[
        pl.BlockSpec((_NB, D), lambda i: (i, 0)),
        pl.BlockSpec((_NB, D), lambda i: (i, 0)),
        pl.BlockSpec((_EB, DE), lambda i: (i, 0)),
        pl.BlockSpec((_EB, DE), lambda i: (i, 0)),
        pl.BlockSpec((D, D), lambda i: (0, 0)),
        pl.BlockSpec((D, D), lambda i: (0, 0)),
        pl.BlockSpec((DE, D), lambda i: (0, 0)),
        pl.BlockSpec((DE, D), lambda i: (0, 0)),
        pl.BlockSpec((1, D), lambda i: (0, 0)),
        pl.BlockSpec((1, D), lambda i: (0, 0)),
    ],
    out_specs=[
        pl.BlockSpec((_NB, D), lambda i: (i, 0)),
        pl.BlockSpec((_NB, D), lambda i: (i, 0)),
        pl.BlockSpec((_EB, D), lambda i: (i, 0)),
        pl.BlockSpec((_EB, D), lambda i: (i, 0)),
    ],
    out_shape=[
        jax.ShapeDtypeStruct((N, D), _F32),
        jax.ShapeDtypeStruct((N, D), _F32),
        jax.ShapeDtypeStruct((E, D), _F32),
        jax.ShapeDtypeStruct((E, D), _F32),
    ],
)


def _sc_body(y0, e0, s0, d0, y1, e1, s1, d1, iota, ones,
             sum0, cnt0, sum1, cnt1,
             acc, iv_s, iv_d, rows_v, emb_v,
             iv_s2, iv_d2, rows_v2, emb_v2, sem, sem2, sg, sg2):
    cid = lax.axis_index("core")
    sid = lax.axis_index("subcore")
    base = pl.multiple_of(sid * 624, 8)
    zero16 = jnp.zeros((16,), _F32)

    def zero_acc():
        @pl.loop(0, CHUNK)
        def _(r):
            for c in range(8):
                rows_v[r, pl.ds(c * 16, 16)] = zero16

        for off in range(0, 640, CHUNK):
            pltpu.sync_copy(iota.at[pl.ds(base + off, CHUNK)], iv_s)
            pltpu.sync_copy(rows_v, acc.at[iv_s])

    def flush(out):
        for off in range(0, 640, CHUNK):
            pltpu.sync_copy(iota.at[pl.ds(base + off, CHUNK)], iv_s)
            pltpu.sync_copy(acc.at[iv_s], rows_v)
            pltpu.sync_copy(rows_v, out.at[pl.ds(base + off, CHUNK)])

    def msg_pass(tbl, emb, srcf, dstf):
        ebase = sid * EPS
        sets = ((iv_s, iv_d, rows_v, emb_v, sem, sg),
                (iv_s2, iv_d2, rows_v2, emb_v2, sem2, sg2))

        def eoff_of(j):
            return pl.multiple_of(ebase + j * CHUNK, 8)

        def start_loads(j, s):
            ivs, ivd, rv, ev, sl, _ = s
            eoff = eoff_of(j)
            pltpu.async_copy(srcf.at[pl.ds(eoff, CHUNK)], ivs, sl)
            pltpu.async_copy(dstf.at[pl.ds(eoff, CHUNK)], ivd, sl)
            pltpu.async_copy(emb.at[pl.ds(eoff, CHUNK)], ev, sl)

        def drain_loads(j, s):
            ivs, ivd, rv, ev, sl, _ = s
            eoff = eoff_of(j)
            pltpu.make_async_copy(srcf.at[pl.ds(eoff, CHUNK)], ivs, sl).wait()
            pltpu.make_async_copy(dstf.at[pl.ds(eoff, CHUNK)], ivd, sl).wait()
            pltpu.make_async_copy(emb.at[pl.ds(eoff, CHUNK)], ev, sl).wait()

        def gather_start(s):
            ivs, _, rv, _, _, sgx = s
            pltpu.async_copy(tbl.at[ivs], rv, sgx)

        def finish(s):
            ivs, ivd, rv, ev, _, sgx = s
            pltpu.make_async_copy(tbl.at[ivs], rv, sgx).wait()

            @pl.loop(0, CHUNK)
            def _(r):
                for c in range(8):
                    slc = pl.ds(c * 16, 16)
                    rv[r, slc] = jnp.maximum(rv[r, slc] + ev[r, slc], 0.0)

            pltpu.sync_copy(rv, acc.at[ivd], add=True)

        a, b = sets
        start_loads(0, a)
        drain_loads(0, a)
        gather_start(a)
        start_loads(1, b)

        @pl.loop(0, NCHUNK // 2)
        def _(p):
            j0 = p * 2
            drain_loads(j0 + 1, b)
            gather_start(b)
            finish(a)
            start_loads(j0 + 2, a)
            drain_loads(j0 + 2, a)
            gather_start(a)
            finish(b)

            @pl.when(j0 + 3 < NCHUNK)
            def _():
                start_loads(j0 + 3, b)

        finish(a)

    def cnt_pass(dstf):
        ebase = sid * EPS
        pltpu.sync_copy(ones, emb_v)

        def eoff_of(j):
            return pl.multiple_of(ebase + j * CHUNK, 8)

        pltpu.async_copy(dstf.at[pl.ds(eoff_of(0), CHUNK)], iv_d, sem)

        @pl.loop(0, NCHUNK // 2)
        def _(p):
            j0 = p * 2
            pltpu.async_copy(dstf.at[pl.ds(eoff_of(j0 + 1), CHUNK)],
                             iv_d2, sem2)
            pltpu.make_async_copy(dstf.at[pl.ds(eoff_of(j0), CHUNK)],
                                  iv_d, sem).wait()
            pltpu.sync_copy(emb_v, acc.at[iv_d], add=True)
            pltpu.async_copy(dstf.at[pl.ds(eoff_of(j0 + 2), CHUNK)],
                             iv_d, sem)
            pltpu.make_async_copy(dstf.at[pl.ds(eoff_of(j0 + 1), CHUNK)],
                                  iv_d2, sem2).wait()
            pltpu.sync_copy(emb_v, acc.at[iv_d2], add=True)

        pltpu.make_async_copy(dstf.at[pl.ds(eoff_of(NCHUNK - 1), CHUNK)],
                              iv_d, sem).wait()
        pltpu.sync_copy(emb_v, acc.at[iv_d], add=True)

    zero_acc()
    plsc.subcore_barrier()

    @pl.when(cid == 0)
    def _():
        msg_pass(y0, e0, s0, d0)

    @pl.when(cid == 1)
    def _():
        msg_pass(y1, e1, s1, d1)

    plsc.subcore_barrier()

    @pl.when(cid == 0)
    def _():
        flush(sum0)

    @pl.when(cid == 1)
    def _():
        flush(sum1)

    plsc.subcore_barrier()
    zero_acc()
    plsc.subcore_barrier()

    @pl.when(cid == 0)
    def _():
        cnt_pass(d0)

    @pl.when(cid == 1)
    def _():
        cnt_pass(d1)

    plsc.subcore_barrier()

    @pl.when(cid == 0)
    def _():
        flush(cnt0)

    @pl.when(cid == 1)
    def _():
        flush(cnt1)


_sc_agg = pl.kernel(
    _sc_body,
    out_type=[
        jax.ShapeDtypeStruct((N, D), _F32),
        jax.ShapeDtypeStruct((N, D), _F32),
        jax.ShapeDtypeStruct((N, D), _F32),
        jax.ShapeDtypeStruct((N, D), _F32),
    ],
    mesh=plsc.VectorSubcoreMesh(core_axis_name="core",
                                subcore_axis_name="subcore"),
    scratch_types=[
        pltpu.VMEM_SHARED((N, D), _F32),
        pltpu.VMEM((CHUNK,), jnp.int32),
        pltpu.VMEM((CHUNK,), jnp.int32),
        pltpu.VMEM((CHUNK, D), _F32),
        pltpu.VMEM((CHUNK, D), _F32),
        pltpu.VMEM((CHUNK,), jnp.int32),
        pltpu.VMEM((CHUNK,), jnp.int32),
        pltpu.VMEM((CHUNK, D), _F32),
        pltpu.VMEM((CHUNK, D), _F32),
        pltpu.SemaphoreType.DMA,
        pltpu.SemaphoreType.DMA,
        pltpu.SemaphoreType.DMA,
        pltpu.SemaphoreType.DMA,
    ],
)


def _final_body(xp, sp, cp, wrp, wcp, xe, se, ce, wre, wce, op, oe):
    def head(x, s, c, wr, wc, o):
        agg = s[...] / jnp.maximum(c[...], 1.0)
        out = jnp.dot(x[...], wr[...], preferred_element_type=_F32) + agg
        o[...] = jnp.dot(jnp.maximum(out, 0.0), wc[...],
                         preferred_element_type=_F32)

    head(xp, sp, cp, wrp, wcp, op)
    head(xe, se, ce, wre, wce, oe)


_FIN_GRID = 10
_FB = N // _FIN_GRID

_final = pl.pallas_call(
    _final_body,
    grid=(_FIN_GRID,),
    in_specs=[
        pl.BlockSpec((_FB, D), lambda i: (i, 0)),
        pl.BlockSpec((_FB, D), lambda i: (i, 0)),
        pl.BlockSpec((_FB, D), lambda i: (i, 0)),
        pl.BlockSpec((D, D), lambda i: (0, 0)),
        pl.BlockSpec((D, D), lambda i: (0, 0)),
        pl.BlockSpec((_FB, D), lambda i: (i, 0)),
        pl.BlockSpec((_FB, D), lambda i: (i, 0)),
        pl.BlockSpec((_FB, D), lambda i: (i, 0)),
        pl.BlockSpec((D, D), lambda i: (0, 0)),
        pl.BlockSpec((D, D), lambda i: (0, 0)),
    ],
    out_specs=[
        pl.BlockSpec((_FB, D), lambda i: (i, 0)),
        pl.BlockSpec((_FB, D), lambda i: (i, 0)),
    ],
    out_shape=[
        jax.ShapeDtypeStruct((N, D), _F32),
        jax.ShapeDtypeStruct((N, D), _F32),
    ],
)


def kernel(x_position, x_equity, edge_index_pe, edge_index_ep,
           edge_attr_pe, edge_attr_ep, W_msg_pe, b_msg_pe, W_msg_ep,
           b_msg_ep, W_root_pos, W_root_eq, W_clf_pos, W_clf_ast):
    wx_pe, we_pe = W_msg_pe[:D], W_msg_pe[D:]
    wx_ep, we_ep = W_msg_ep[:D], W_msg_ep[D:]

    y_pe, y_ep, emb_pe, emb_ep = _prep(
        x_position, x_equity, edge_attr_pe, edge_attr_ep,
        wx_pe, wx_ep, we_pe, we_ep,
        b_msg_pe.reshape(1, D), b_msg_ep.reshape(1, D))

    iota = jnp.arange(N, dtype=jnp.int32)
    ones = jnp.ones((CHUNK, D), _F32)

    sum0, cnt0, sum1, cnt1 = _sc_agg(
        y_pe, emb_pe, edge_index_pe[0], edge_index_pe[1],
        y_ep, emb_ep, edge_index_ep[0], edge_index_ep[1],
        iota, ones)

    wc_pos = jnp.pad(W_clf_pos, ((0, 0), (0, D - W_clf_pos.shape[1])))
    wc_ast = jnp.pad(W_clf_ast, ((0, 0), (0, D - W_clf_ast.shape[1])))

    p_pos, p_eq = _final(x_position, sum1, cnt1, W_root_pos, wc_pos,
                         x_equity, sum0, cnt0, W_root_eq, wc_ast)

    nc = W_clf_pos.shape[1]
    return jnp.concatenate([p_pos[:, :nc], p_eq[:, :nc]], axis=0)
